# speedup vs baseline: 43.1163x; 43.1163x over previous
import functools

import jax
import jax.numpy as jnp
from jax.experimental import pallas as pl
from jax.experimental.pallas import tpu as pltpu


def _point_mean_kernel(x_ref, w0_ref, b0_ref, w1_ref, b1_ref, o_ref, acc_ref,
                       *, inv_l, n_l):
    l = pl.program_id(1)

    @pl.when(l == 0)
    def _():
        acc_ref[...] = jnp.zeros_like(acc_ref)

    x = x_ref[0]
    h1 = jnp.dot(w0_ref[...], x, preferred_element_type=jnp.float32)
    h1 = jnp.maximum(h1 + b0_ref[...], 0.0)
    h2 = jnp.dot(w1_ref[...], h1, preferred_element_type=jnp.float32)
    h2 = jnp.maximum(h2 + b1_ref[...], 0.0)
    ones = jnp.ones((1, h2.shape[1]), jnp.float32)
    part = jax.lax.dot_general(ones, h2, (((1,), (1,)), ((), ())),
                               preferred_element_type=jnp.float32)
    acc_ref[...] += part

    @pl.when(l == n_l - 1)
    def _():
        o_ref[...] = (acc_ref[...] * inv_l).reshape(o_ref.shape)


def _head_kernel(m_ref, w0_ref, b0_ref, w1_ref, b1_ref, w2_ref, b2_ref, o_ref):
    g = jnp.dot(m_ref[...], w0_ref[...], preferred_element_type=jnp.float32)
    g = jnp.maximum(g + b0_ref[...], 0.0)
    g = jnp.dot(g, w1_ref[...], preferred_element_type=jnp.float32)
    g = jnp.maximum(g + b1_ref[...], 0.0)
    o_ref[...] = (jnp.dot(g, w2_ref[...], preferred_element_type=jnp.float32)
                  + b2_ref[...])


def _pick_tl(L):
    for tl in (16384, 8192, 4096, 2048, 1024, 512, 256, 128):
        if L % tl == 0:
            return tl
    return L


def kernel(point_cloud, lw0, lb0, lw1, lb1, gw0, gb0, gw1, gb1, gw2, gb2):
    B, C, L = point_cloud.shape
    H1 = lw0.shape[1]
    H2 = lw1.shape[1]

    tl = _pick_tl(L)
    n_l = L // tl

    w0t = lw0.T
    b0c = lb0.reshape(H1, 1)
    w1t = lw1.T
    b1c = lb1.reshape(H2, 1)

    means = pl.pallas_call(
        functools.partial(_point_mean_kernel, inv_l=1.0 / L, n_l=n_l),
        out_shape=jax.ShapeDtypeStruct((B, 1, H2), jnp.float32),
        grid=(B, n_l),
        in_specs=[
            pl.BlockSpec((1, C, tl), lambda b, l: (b, 0, l)),
            pl.BlockSpec((H1, C), lambda b, l: (0, 0)),
            pl.BlockSpec((H1, 1), lambda b, l: (0, 0)),
            pl.BlockSpec((H2, H1), lambda b, l: (0, 0)),
            pl.BlockSpec((H2, 1), lambda b, l: (0, 0)),
        ],
        out_specs=pl.BlockSpec((1, 1, H2), lambda b, l: (b, 0, 0)),
        scratch_shapes=[pltpu.VMEM((1, H2), jnp.float32)],
        compiler_params=pltpu.CompilerParams(
            dimension_semantics=("parallel", "arbitrary"),
            vmem_limit_bytes=96 * 1024 * 1024,
        ),
    )(point_cloud, w0t, b0c, w1t, b1c)

    m = means.reshape(B, H2)

    out = pl.pallas_call(
        _head_kernel,
        out_shape=jax.ShapeDtypeStruct((B, 1), jnp.float32),
    )(m, gw0, gb0.reshape(1, -1), gw1, gb1.reshape(1, -1),
      gw2, gb2.reshape(1, 1))
    return out
